# baseline (speedup 1.0000x reference)
import jax
import jax.numpy as jnp
from jax.experimental import pallas as pl

B = 64
NPG = 256
N = B * NPG
E = 262144
NODE_FEAT = 16
HID = 256
HEADS = 8
HD = HID // HEADS
LAYERS = 2
GATE_TYPES = 32
EDGE_FEAT = 16
FF = 1024
RW_K = 8
MAX_DEG = 256
SCALE = HD ** -0.5


def _layernorm(h, s, b):
    m = jnp.mean(h, axis=-1, keepdims=True)
    v = jnp.var(h, axis=-1, keepdims=True)
    return (h - m) / jnp.sqrt(v + 1e-5) * s + b


def _identity_kernel(x_ref, o_ref):
    o_ref[...] = x_ref[...]


def kernel(x, edge_index, edge_attr, edge_gate_type, batch, params):
    src = edge_index[0]
    dst = edge_index[1]
    g = src // NPG
    ls = src % NPG
    ld = dst % NPG
    h = x @ params['Win'] + params['bin']
    deg = jnp.zeros((N,), jnp.float32).at[dst].add(1.0)
    deg_idx = jnp.clip(deg.astype(jnp.int32), 0, MAX_DEG - 1)
    pe = jnp.take(params['deg_emb'], deg_idx, axis=0)
    adj = jnp.zeros((B, NPG, NPG), jnp.float32).at[g, ls, ld].add(1.0)
    adj = adj + jnp.swapaxes(adj, 1, 2)
    adj = (adj > 0).astype(jnp.float32)
    degg = jnp.sum(adj, axis=2)
    deg_inv = jnp.where(degg > 0, 1.0 / jnp.where(degg > 0, degg, 1.0), 0.0)
    transition = adj * deg_inv[:, None, :]
    power = jnp.broadcast_to(jnp.eye(NPG, dtype=jnp.float32), (B, NPG, NPG))
    diags = []
    for _ in range(RW_K):
        power = power @ transition
        diags.append(jnp.diagonal(power, axis1=1, axis2=2))
    rw_pe = jnp.stack(diags, axis=-1).reshape(N, RW_K)
    pe = pe + rw_pe @ params['rwW'] + params['rwb']
    h = h + pe
    for l in range(LAYERS):
        Q = (h @ params['Wq'][l] + params['bq'][l]).reshape(B, NPG, HEADS, HD).transpose(0, 2, 1, 3)
        K = (h @ params['Wk'][l] + params['bk'][l]).reshape(B, NPG, HEADS, HD).transpose(0, 2, 1, 3)
        V = (h @ params['Wv'][l] + params['bv'][l]).reshape(B, NPG, HEADS, HD).transpose(0, 2, 1, 3)
        scores = (Q @ jnp.swapaxes(K, -1, -2)) * SCALE
        eb = jnp.take(params['gate_emb'][l], edge_gate_type, axis=0) + edge_attr @ params['ebW'][l] + params['ebb'][l]
        bias = jnp.zeros((B, HEADS, NPG, NPG), jnp.float32).at[g, :, ls, ld].add(eb)
        scores = scores + bias
        attn = jax.nn.softmax(scores, axis=-1)
        out = (attn @ V).transpose(0, 2, 1, 3).reshape(N, HID)
        h = _layernorm(h + out @ params['Wo'][l] + params['bo'][l], params['ln1s'][l], params['ln1b'][l])
        ff = jax.nn.relu(h @ params['W1'][l] + params['b1'][l]) @ params['W2'][l] + params['b2'][l]
        h = _layernorm(h + ff, params['ln2s'][l], params['ln2b'][l])
    h = pl.pallas_call(
        _identity_kernel,
        out_shape=jax.ShapeDtypeStruct(h.shape, h.dtype),
    )(h)
    return h


# speedup vs baseline: 2.0429x; 2.0429x over previous
import jax
import jax.numpy as jnp
from jax import lax
from jax.experimental import pallas as pl
from jax.experimental.pallas import tpu as pltpu

B = 64
NPG = 256
N = B * NPG
E = 262144
NODE_FEAT = 16
HID = 256
HEADS = 8
HD = HID // HEADS
LAYERS = 2
GATE_TYPES = 32
EDGE_FEAT = 16
FF = 1024
RW_K = 8
MAX_DEG = 256
SCALE = HD ** -0.5

EB_BLK = 8192


def _edge_feat_kernel(gt_ref, ea_ref, gate_cat_ref, ebw_cat_ref, ebb_cat_ref, out_ref):
    gt = gt_ref[...]
    oh = (gt == lax.broadcasted_iota(jnp.int32, (EB_BLK, GATE_TYPES), 1)).astype(jnp.float32)
    out_ref[...] = (oh @ gate_cat_ref[...]
                    + ea_ref[...] @ ebw_cat_ref[...]
                    + ebb_cat_ref[...])


def _rw_pe_kernel(adj_ref, adjT_ref, out_ref):
    a = adj_ref[0] + adjT_ref[0]
    a = (a > 0.0).astype(jnp.float32)
    degg = jnp.sum(a, axis=1, keepdims=True)
    deg_inv = jnp.where(degg > 0.0, 1.0 / jnp.where(degg > 0.0, degg, 1.0), 0.0)
    t = a * deg_inv.reshape(1, NPG)
    eye = (lax.broadcasted_iota(jnp.int32, (NPG, NPG), 0)
           == lax.broadcasted_iota(jnp.int32, (NPG, NPG), 1)).astype(jnp.float32)
    power = eye
    diags = []
    for _ in range(RW_K):
        power = lax.dot_general(power, t, (((1,), (0,)), ((), ())),
                                preferred_element_type=jnp.float32)
        diags.append(jnp.sum(power * eye, axis=1, keepdims=True))
    out_ref[0] = jnp.concatenate(diags, axis=1)


def _layernorm(h, s, b):
    m = jnp.mean(h, axis=-1, keepdims=True)
    v = jnp.mean((h - m) * (h - m), axis=-1, keepdims=True)
    return (h - m) / jnp.sqrt(v + 1e-5) * s + b


def _encoder_kernel(x_ref, deg_ref, rw_ref, bias_ref,
                    win_ref, bin_ref, demb_ref, rww_ref, rwb_ref,
                    wq_ref, bq_ref, wk_ref, bk_ref, wv_ref, bv_ref,
                    wo_ref, bo_ref, ln1s_ref, ln1b_ref, ln2s_ref, ln2b_ref,
                    w1_ref, b1_ref, w2_ref, b2_ref, out_ref):
    x = x_ref[0]
    deg = deg_ref[0]
    rw = rw_ref[0]
    h = x @ win_ref[...] + bin_ref[...]
    deg_oh = (deg == lax.broadcasted_iota(jnp.int32, (NPG, MAX_DEG), 1)).astype(jnp.float32)
    h = h + deg_oh @ demb_ref[...]
    h = h + lax.dot_general(rw, rww_ref[...], (((1,), (0,)), ((), ())),
                            preferred_element_type=jnp.float32) + rwb_ref[...]
    for l in range(LAYERS):
        q = h @ wq_ref[l] + bq_ref[l]
        k = h @ wk_ref[l] + bk_ref[l]
        v = h @ wv_ref[l] + bv_ref[l]
        outs = []
        for hh in range(HEADS):
            qh = q[:, hh * HD:(hh + 1) * HD]
            kh = k[:, hh * HD:(hh + 1) * HD]
            vh = v[:, hh * HD:(hh + 1) * HD]
            s = lax.dot_general(qh, kh, (((1,), (1,)), ((), ())),
                                preferred_element_type=jnp.float32) * SCALE
            s = s + bias_ref[0, l * HEADS + hh]
            m = jnp.max(s, axis=1, keepdims=True)
            p = jnp.exp(s - m)
            p = p / jnp.sum(p, axis=1, keepdims=True)
            outs.append(lax.dot_general(p, vh, (((1,), (0,)), ((), ())),
                                        preferred_element_type=jnp.float32))
        attn = jnp.concatenate(outs, axis=1)
        h = _layernorm(h + attn @ wo_ref[l] + bo_ref[l], ln1s_ref[l], ln1b_ref[l])
        ff = jnp.maximum(h @ w1_ref[l] + b1_ref[l], 0.0) @ w2_ref[l] + b2_ref[l]
        h = _layernorm(h + ff, ln2s_ref[l], ln2b_ref[l])
    out_ref[0] = h


def _whole(shape):
    nd = len(shape)
    return pl.BlockSpec(shape, lambda b, _nd=nd: (0,) * _nd)


def kernel(x, edge_index, edge_attr, edge_gate_type, batch, params):
    p = params
    src = edge_index[0]
    dst = edge_index[1]
    g = src // NPG
    ls = src % NPG
    ld = dst % NPG

    gate_cat = jnp.concatenate([p['gate_emb'][0], p['gate_emb'][1]], axis=1)
    ebw_cat = jnp.concatenate([p['ebW'][0], p['ebW'][1]], axis=1)
    ebb_cat = jnp.concatenate([p['ebb'][0], p['ebb'][1]], axis=0).reshape(1, 2 * HEADS)
    gt2 = edge_gate_type.reshape(E, 1)
    eb = pl.pallas_call(
        _edge_feat_kernel,
        grid=(E // EB_BLK,),
        in_specs=[
            pl.BlockSpec((EB_BLK, 1), lambda i: (i, 0)),
            pl.BlockSpec((EB_BLK, EDGE_FEAT), lambda i: (i, 0)),
            _whole((GATE_TYPES, 2 * HEADS)),
            _whole((EDGE_FEAT, 2 * HEADS)),
            _whole((1, 2 * HEADS)),
        ],
        out_specs=pl.BlockSpec((EB_BLK, 2 * HEADS), lambda i: (i, 0)),
        out_shape=jax.ShapeDtypeStruct((E, 2 * HEADS), jnp.float32),
    )(gt2, edge_attr, gate_cat, ebw_cat, ebb_cat)

    deg = jnp.zeros((N,), jnp.float32).at[dst].add(1.0)
    deg_idx = jnp.clip(deg.astype(jnp.int32), 0, MAX_DEG - 1).reshape(B, NPG, 1)
    adjc = jnp.zeros((B, NPG, NPG), jnp.float32).at[g, ls, ld].add(1.0)
    adjcT = jnp.swapaxes(adjc, 1, 2)
    bias = jnp.zeros((B, LAYERS * HEADS, NPG, NPG), jnp.float32).at[g, :, ls, ld].add(eb)

    rw = pl.pallas_call(
        _rw_pe_kernel,
        grid=(B,),
        in_specs=[
            pl.BlockSpec((1, NPG, NPG), lambda b: (b, 0, 0)),
            pl.BlockSpec((1, NPG, NPG), lambda b: (b, 0, 0)),
        ],
        out_specs=pl.BlockSpec((1, NPG, RW_K), lambda b: (b, 0, 0)),
        out_shape=jax.ShapeDtypeStruct((B, NPG, RW_K), jnp.float32),
    )(adjc, adjcT)

    x3 = x.reshape(B, NPG, NODE_FEAT)
    out = pl.pallas_call(
        _encoder_kernel,
        grid=(B,),
        in_specs=[
            pl.BlockSpec((1, NPG, NODE_FEAT), lambda b: (b, 0, 0)),
            pl.BlockSpec((1, NPG, 1), lambda b: (b, 0, 0)),
            pl.BlockSpec((1, NPG, RW_K), lambda b: (b, 0, 0)),
            pl.BlockSpec((1, LAYERS * HEADS, NPG, NPG), lambda b: (b, 0, 0, 0)),
            _whole((NODE_FEAT, HID)),
            _whole((1, HID)),
            _whole((MAX_DEG, HID)),
            _whole((RW_K, HID)),
            _whole((1, HID)),
            _whole((LAYERS, HID, HID)),
            _whole((LAYERS, 1, HID)),
            _whole((LAYERS, HID, HID)),
            _whole((LAYERS, 1, HID)),
            _whole((LAYERS, HID, HID)),
            _whole((LAYERS, 1, HID)),
            _whole((LAYERS, HID, HID)),
            _whole((LAYERS, 1, HID)),
            _whole((LAYERS, 1, HID)),
            _whole((LAYERS, 1, HID)),
            _whole((LAYERS, 1, HID)),
            _whole((LAYERS, 1, HID)),
            _whole((LAYERS, HID, FF)),
            _whole((LAYERS, 1, FF)),
            _whole((LAYERS, FF, HID)),
            _whole((LAYERS, 1, HID)),
        ],
        out_specs=pl.BlockSpec((1, NPG, HID), lambda b: (b, 0, 0)),
        out_shape=jax.ShapeDtypeStruct((B, NPG, HID), jnp.float32),
    )(x3, deg_idx, rw, bias,
      p['Win'], p['bin'].reshape(1, HID), p['deg_emb'], p['rwW'], p['rwb'].reshape(1, HID),
      p['Wq'], p['bq'].reshape(LAYERS, 1, HID), p['Wk'], p['bk'].reshape(LAYERS, 1, HID),
      p['Wv'], p['bv'].reshape(LAYERS, 1, HID), p['Wo'], p['bo'].reshape(LAYERS, 1, HID),
      p['ln1s'].reshape(LAYERS, 1, HID), p['ln1b'].reshape(LAYERS, 1, HID),
      p['ln2s'].reshape(LAYERS, 1, HID), p['ln2b'].reshape(LAYERS, 1, HID),
      p['W1'], p['b1'].reshape(LAYERS, 1, FF), p['W2'], p['b2'].reshape(LAYERS, 1, HID))
    return out.reshape(N, HID)


# speedup vs baseline: 2.0734x; 1.0149x over previous
import functools
import jax
import jax.numpy as jnp
from jax import lax
from jax.experimental import pallas as pl
from jax.experimental.pallas import tpu as pltpu
from jax.experimental.pallas import tpu_sc as plsc

B = 64
NPG = 256
N = B * NPG
E = 262144
NODE_FEAT = 16
HID = 256
HEADS = 8
HD = HID // HEADS
LAYERS = 2
GATE_TYPES = 32
EDGE_FEAT = 16
FF = 1024
RW_K = 8
MAX_DEG = 256
SCALE = HD ** -0.5

NSLOT = LAYERS * HEADS
GSIZE = NPG * NPG
EB_BLK = 8192

NC, NS = 2, 16
NW = NC * NS
EPW = E // NW
CHA = 1024
SUB = 128
CH = 1024
CHP = CH + 16
EPAD = E + 4096

_MESH = plsc.VectorSubcoreMesh(core_axis_name="c", subcore_axis_name="s",
                               num_cores=NC, num_subcores=NS)


def _edge_rec_kernel(gt_ref, pos_ref, ea_ref, gate_cat_ref, ebw_cat_ref, ebb_cat_ref, out_ref):
    gt = gt_ref[...]
    oh = (gt == lax.broadcasted_iota(jnp.int32, (EB_BLK, GATE_TYPES), 1)).astype(jnp.float32)
    eb = (oh @ gate_cat_ref[...]
          + ea_ref[...] @ ebw_cat_ref[...]
          + ebb_cat_ref[...])
    posf = lax.bitcast_convert_type(pos_ref[...], jnp.float32)
    out_ref[...] = jnp.concatenate(
        [posf, eb, jnp.zeros((EB_BLK, 32 - 1 - NSLOT), jnp.float32)], axis=1)


def _rw_pe_kernel(adj_ref, out_ref):
    a = (adj_ref[0] > 0.0).astype(jnp.float32)
    degg = jnp.sum(a, axis=1, keepdims=True)
    deg_inv = jnp.where(degg > 0.0, 1.0 / jnp.where(degg > 0.0, degg, 1.0), 0.0)
    t = a * deg_inv.reshape(1, NPG)
    eye = (lax.broadcasted_iota(jnp.int32, (NPG, NPG), 0)
           == lax.broadcasted_iota(jnp.int32, (NPG, NPG), 1)).astype(jnp.float32)
    power = eye
    diags = []
    for _ in range(RW_K):
        power = lax.dot_general(power, t, (((1,), (0,)), ((), ())),
                                preferred_element_type=jnp.float32)
        diags.append(jnp.sum(power * eye, axis=1, keepdims=True))
    out_ref[0] = jnp.concatenate(diags, axis=1)


def _layernorm(h, s, b):
    m = jnp.mean(h, axis=-1, keepdims=True)
    v = jnp.mean((h - m) * (h - m), axis=-1, keepdims=True)
    return (h - m) / jnp.sqrt(v + 1e-5) * s + b


def _encoder_kernel(x_ref, deg_ref, rw_ref, bias_ref,
                    win_ref, bin_ref, demb_ref, rww_ref, rwb_ref,
                    wq_ref, bq_ref, wk_ref, bk_ref, wv_ref, bv_ref,
                    wo_ref, bo_ref, ln1s_ref, ln1b_ref, ln2s_ref, ln2b_ref,
                    w1_ref, b1_ref, w2_ref, b2_ref, out_ref):
    x = x_ref[0]
    deg = deg_ref[0]
    rw = rw_ref[0]
    h = x @ win_ref[...] + bin_ref[...]
    deg_oh = (deg == lax.broadcasted_iota(jnp.int32, (NPG, MAX_DEG), 1)).astype(jnp.float32)
    h = h + deg_oh @ demb_ref[...]
    h = h + lax.dot_general(rw, rww_ref[...], (((1,), (0,)), ((), ())),
                            preferred_element_type=jnp.float32) + rwb_ref[...]
    for l in range(LAYERS):
        q = h @ wq_ref[l] + bq_ref[l]
        k = h @ wk_ref[l] + bk_ref[l]
        v = h @ wv_ref[l] + bv_ref[l]
        outs = []
        for hh in range(HEADS):
            qh = q[:, hh * HD:(hh + 1) * HD]
            kh = k[:, hh * HD:(hh + 1) * HD]
            vh = v[:, hh * HD:(hh + 1) * HD]
            s = lax.dot_general(qh, kh, (((1,), (1,)), ((), ())),
                                preferred_element_type=jnp.float32) * SCALE
            s = s + bias_ref[0, l * HEADS + hh]
            m = jnp.max(s, axis=1, keepdims=True)
            p = jnp.exp(s - m)
            p = p / jnp.sum(p, axis=1, keepdims=True)
            outs.append(lax.dot_general(p, vh, (((1,), (0,)), ((), ())),
                                        preferred_element_type=jnp.float32))
        attn = jnp.concatenate(outs, axis=1)
        h = _layernorm(h + attn @ wo_ref[l] + bo_ref[l], ln1s_ref[l], ln1b_ref[l])
        ff = jnp.maximum(h @ w1_ref[l] + b1_ref[l], 0.0) @ w2_ref[l] + b2_ref[l]
        h = _layernorm(h + ff, ln2s_ref[l], ln2b_ref[l])
    out_ref[0] = h


def _wid():
    return lax.axis_index("s") * NC + lax.axis_index("c")


def _scal(offv, i):
    return offv[pl.ds(i, 16)][0]


def _zero_ref(ref, nwords):
    z = jnp.zeros((16,), jnp.float32)

    def body(j, _):
        ref[pl.ds(j * 16, 16)] = z
        return 0

    lax.fori_loop(0, nwords // 16, body, 0)


def _reorder_body(rec_hbm, order_hbm, srec_hbm, ordv, recv, sem):
    w = _wid()

    def chunk(c, _):
        base = pl.multiple_of(w * EPW + c * CHA, CHA)
        pltpu.sync_copy(order_hbm.at[pl.ds(pl.multiple_of(base // SUB, 8), CHA // SUB)], ordv)
        copies = []
        for k in range(CHA // SUB):
            copies.append(pltpu.async_copy(
                rec_hbm.at[ordv.at[k]], recv.at[pl.ds(k * SUB, SUB)], sem))
        for cp in copies:
            cp.wait()
        pltpu.sync_copy(recv, srec_hbm.at[pl.ds(base, CHA)])
        return 0

    lax.fori_loop(0, EPW // CHA, chunk, 0)


def _bias_body(srec_flat_hbm, off_hbm, bias_hbm, dest, chv, offv):
    w = _wid()
    pltpu.sync_copy(off_hbm, offv)
    lanes16 = lax.iota(jnp.int32, 16)

    def task(t, _):
        tid = w + NW * t
        g = tid // NSLOT
        s = tid % NSLOT
        o0 = _scal(offv, g)
        o1 = _scal(offv, g + 1)
        cnt = o1 - o0
        _zero_ref(dest, GSIZE)
        nch = (cnt + CH - 1) // CH

        def cbody(c, _):
            lo = o0 + c * CH
            hi = jnp.minimum(lo + CH, o1)
            astart = pl.multiple_of((lo // 8) * 8, 8)
            pltpu.sync_copy(srec_flat_hbm.at[pl.ds(pl.multiple_of(astart * 32, 256), CHP * 32)], chv)

            def vbody(j, _):
                gidx = lanes16 + (astart + j * 16)
                mask = (gidx >= lo) & (gidx < hi)
                fidx = (lanes16 + j * 16) * 32
                pv = plsc.bitcast(plsc.load_gather(chv, [fidx]), jnp.int32)
                vv = plsc.load_gather(chv, [fidx + (s + 1)])
                plsc.addupdate_scatter(dest, [pv], vv, mask=mask)
                return 0

            lax.fori_loop(0, CHP // 16, vbody, 0)
            return 0

        lax.fori_loop(0, nch, cbody, 0)
        pltpu.sync_copy(dest, bias_hbm.at[g, s])
        return 0

    lax.fori_loop(0, B * NSLOT // NW, task, 0)


def _adj_deg_body(srec_flat_hbm, off_hbm, adj_hbm, deg_hbm, dest, degd, chv, offv):
    w = _wid()
    pltpu.sync_copy(off_hbm, offv)
    lanes16 = lax.iota(jnp.int32, 16)
    ones = jnp.ones((16,), jnp.float32)

    def task(t, _):
        g = w + NW * t
        o0 = _scal(offv, g)
        o1 = _scal(offv, g + 1)
        cnt = o1 - o0
        _zero_ref(dest, GSIZE)
        _zero_ref(degd, NPG)
        nch = (cnt + CH - 1) // CH

        def cbody(c, _):
            lo = o0 + c * CH
            hi = jnp.minimum(lo + CH, o1)
            astart = pl.multiple_of((lo // 8) * 8, 8)
            pltpu.sync_copy(srec_flat_hbm.at[pl.ds(pl.multiple_of(astart * 32, 256), CHP * 32)], chv)

            def vbody(j, _):
                gidx = lanes16 + (astart + j * 16)
                mask = (gidx >= lo) & (gidx < hi)
                fidx = (lanes16 + j * 16) * 32
                pv = plsc.bitcast(plsc.load_gather(chv, [fidx]), jnp.int32)
                ptv = ((pv & 255) << 8) | (pv >> 8)
                plsc.addupdate_scatter(dest, [pv], ones, mask=mask)
                plsc.addupdate_scatter(dest, [ptv], ones, mask=mask)
                plsc.addupdate_scatter(degd, [pv & 255], ones, mask=mask)
                return 0

            lax.fori_loop(0, CHP // 16, vbody, 0)
            return 0

        lax.fori_loop(0, nch, cbody, 0)
        pltpu.sync_copy(dest, adj_hbm.at[g])
        pltpu.sync_copy(degd, deg_hbm.at[g])
        return 0

    lax.fori_loop(0, B // NW, task, 0)


_reorder_call = functools.partial(
    pl.kernel,
    out_type=jax.ShapeDtypeStruct((EPAD, 32), jnp.float32),
    mesh=_MESH,
    compiler_params=pltpu.CompilerParams(use_tc_tiling_on_sc=False, needs_layout_passes=False),
    scratch_types=[
        pltpu.VMEM((CHA // SUB, SUB), jnp.int32),
        pltpu.VMEM((CHA, 32), jnp.float32),
        pltpu.SemaphoreType.DMA,
    ],
)(_reorder_body)

_bias_call = functools.partial(
    pl.kernel,
    out_type=jax.ShapeDtypeStruct((B, NSLOT, GSIZE), jnp.float32),
    mesh=_MESH,
    compiler_params=pltpu.CompilerParams(use_tc_tiling_on_sc=False, needs_layout_passes=False),
    scratch_types=[
        pltpu.VMEM((GSIZE,), jnp.float32),
        pltpu.VMEM((CHP * 32,), jnp.float32),
        pltpu.VMEM((80,), jnp.int32),
    ],
)(_bias_body)

_adj_deg_call = functools.partial(
    pl.kernel,
    out_type=(jax.ShapeDtypeStruct((B, GSIZE), jnp.float32),
              jax.ShapeDtypeStruct((B, NPG), jnp.float32)),
    mesh=_MESH,
    compiler_params=pltpu.CompilerParams(use_tc_tiling_on_sc=False, needs_layout_passes=False),
    scratch_types=[
        pltpu.VMEM((GSIZE,), jnp.float32),
        pltpu.VMEM((NPG,), jnp.float32),
        pltpu.VMEM((CHP * 32,), jnp.float32),
        pltpu.VMEM((80,), jnp.int32),
    ],
)(_adj_deg_body)


def _whole(shape):
    nd = len(shape)
    return pl.BlockSpec(shape, lambda b, _nd=nd: (0,) * _nd)


def kernel(x, edge_index, edge_attr, edge_gate_type, batch, params):
    p = params
    src = edge_index[0]
    dst = edge_index[1]
    g = src // NPG
    ls = src % NPG
    ld = dst % NPG
    pos = ls * NPG + ld

    gate_cat = jnp.concatenate([p['gate_emb'][0], p['gate_emb'][1]], axis=1)
    ebw_cat = jnp.concatenate([p['ebW'][0], p['ebW'][1]], axis=1)
    ebb_cat = jnp.concatenate([p['ebb'][0], p['ebb'][1]], axis=0).reshape(1, NSLOT)
    rec = pl.pallas_call(
        _edge_rec_kernel,
        grid=(E // EB_BLK,),
        in_specs=[
            pl.BlockSpec((EB_BLK, 1), lambda i: (i, 0)),
            pl.BlockSpec((EB_BLK, 1), lambda i: (i, 0)),
            pl.BlockSpec((EB_BLK, EDGE_FEAT), lambda i: (i, 0)),
            _whole((GATE_TYPES, NSLOT)),
            _whole((EDGE_FEAT, NSLOT)),
            _whole((1, NSLOT)),
        ],
        out_specs=pl.BlockSpec((EB_BLK, 32), lambda i: (i, 0)),
        out_shape=jax.ShapeDtypeStruct((E, 32), jnp.float32),
    )(edge_gate_type.reshape(E, 1), pos.reshape(E, 1), edge_attr,
      gate_cat, ebw_cat, ebb_cat)

    gpos = g * GSIZE + pos
    sorted_gpos, order = lax.sort((gpos, jnp.arange(E, dtype=jnp.int32)), num_keys=1)
    off = jnp.searchsorted(sorted_gpos, jnp.arange(B + 1, dtype=jnp.int32) * GSIZE
                           ).astype(jnp.int32)
    off = jnp.concatenate([off, jnp.zeros((80 - (B + 1),), jnp.int32)])

    srec = _reorder_call(rec, order.reshape(E // SUB, SUB))
    srec_flat = srec.reshape(EPAD * 32)
    bias = _bias_call(srec_flat, off)
    adjs, degc = _adj_deg_call(srec_flat, off)

    deg_idx = jnp.clip(degc.astype(jnp.int32), 0, MAX_DEG - 1).reshape(B, NPG, 1)

    rw = pl.pallas_call(
        _rw_pe_kernel,
        grid=(B,),
        in_specs=[pl.BlockSpec((1, NPG, NPG), lambda b: (b, 0, 0))],
        out_specs=pl.BlockSpec((1, NPG, RW_K), lambda b: (b, 0, 0)),
        out_shape=jax.ShapeDtypeStruct((B, NPG, RW_K), jnp.float32),
    )(adjs.reshape(B, NPG, NPG))

    x3 = x.reshape(B, NPG, NODE_FEAT)
    bias4 = bias.reshape(B, NSLOT, NPG, NPG)
    out = pl.pallas_call(
        _encoder_kernel,
        grid=(B,),
        in_specs=[
            pl.BlockSpec((1, NPG, NODE_FEAT), lambda b: (b, 0, 0)),
            pl.BlockSpec((1, NPG, 1), lambda b: (b, 0, 0)),
            pl.BlockSpec((1, NPG, RW_K), lambda b: (b, 0, 0)),
            pl.BlockSpec((1, NSLOT, NPG, NPG), lambda b: (b, 0, 0, 0)),
            _whole((NODE_FEAT, HID)),
            _whole((1, HID)),
            _whole((MAX_DEG, HID)),
            _whole((RW_K, HID)),
            _whole((1, HID)),
            _whole((LAYERS, HID, HID)),
            _whole((LAYERS, 1, HID)),
            _whole((LAYERS, HID, HID)),
            _whole((LAYERS, 1, HID)),
            _whole((LAYERS, HID, HID)),
            _whole((LAYERS, 1, HID)),
            _whole((LAYERS, HID, HID)),
            _whole((LAYERS, 1, HID)),
            _whole((LAYERS, 1, HID)),
            _whole((LAYERS, 1, HID)),
            _whole((LAYERS, 1, HID)),
            _whole((LAYERS, 1, HID)),
            _whole((LAYERS, HID, FF)),
            _whole((LAYERS, 1, FF)),
            _whole((LAYERS, FF, HID)),
            _whole((LAYERS, 1, HID)),
        ],
        out_specs=pl.BlockSpec((1, NPG, HID), lambda b: (b, 0, 0)),
        out_shape=jax.ShapeDtypeStruct((B, NPG, HID), jnp.float32),
    )(x3, deg_idx, rw, bias4,
      p['Win'], p['bin'].reshape(1, HID), p['deg_emb'], p['rwW'], p['rwb'].reshape(1, HID),
      p['Wq'], p['bq'].reshape(LAYERS, 1, HID), p['Wk'], p['bk'].reshape(LAYERS, 1, HID),
      p['Wv'], p['bv'].reshape(LAYERS, 1, HID), p['Wo'], p['bo'].reshape(LAYERS, 1, HID),
      p['ln1s'].reshape(LAYERS, 1, HID), p['ln1b'].reshape(LAYERS, 1, HID),
      p['ln2s'].reshape(LAYERS, 1, HID), p['ln2b'].reshape(LAYERS, 1, HID),
      p['W1'], p['b1'].reshape(LAYERS, 1, FF), p['W2'], p['b2'].reshape(LAYERS, 1, HID))
    return out.reshape(N, HID)


# speedup vs baseline: 2.5684x; 1.2388x over previous
import functools
import jax
import jax.numpy as jnp
from jax import lax
from jax.experimental import pallas as pl
from jax.experimental.pallas import tpu as pltpu
from jax.experimental.pallas import tpu_sc as plsc

B = 64
NPG = 256
N = B * NPG
E = 262144
NODE_FEAT = 16
HID = 256
HEADS = 8
HD = HID // HEADS
LAYERS = 2
GATE_TYPES = 32
EDGE_FEAT = 16
FF = 1024
RW_K = 8
MAX_DEG = 256
SCALE = HD ** -0.5

NSLOT = LAYERS * HEADS
GSIZE = NPG * NPG
EB_BLK = 8192

NC, NS = 2, 16
NW = NC * NS
EPW = E // NW
CHA = 1024
SUB = 128
CH = 2048
CHP = CH + 16
EPAD = E + 4096

_MESH = plsc.VectorSubcoreMesh(core_axis_name="c", subcore_axis_name="s",
                               num_cores=NC, num_subcores=NS)


def _edge_rec_kernel(gt_ref, pos_ref, ea_ref, gate_cat_ref, ebw_cat_ref, ebb_cat_ref, out_ref):
    gt = gt_ref[...]
    oh = (gt == lax.broadcasted_iota(jnp.int32, (EB_BLK, GATE_TYPES), 1)).astype(jnp.float32)
    eb = (oh @ gate_cat_ref[...]
          + ea_ref[...] @ ebw_cat_ref[...]
          + ebb_cat_ref[...])
    posf = lax.bitcast_convert_type(pos_ref[...], jnp.float32)
    out_ref[...] = jnp.concatenate(
        [posf, eb, jnp.zeros((EB_BLK, 32 - 1 - NSLOT), jnp.float32)], axis=1)


def _rw_pe_kernel(adj_ref, out_ref):
    a = (adj_ref[0] > 0.0).astype(jnp.float32)
    degg = jnp.sum(a, axis=1, keepdims=True)
    deg_inv = jnp.where(degg > 0.0, 1.0 / jnp.where(degg > 0.0, degg, 1.0), 0.0)
    t = a * deg_inv.reshape(1, NPG)
    eye = (lax.broadcasted_iota(jnp.int32, (NPG, NPG), 0)
           == lax.broadcasted_iota(jnp.int32, (NPG, NPG), 1)).astype(jnp.float32)
    power = eye
    diags = []
    for _ in range(RW_K):
        power = lax.dot_general(power, t, (((1,), (0,)), ((), ())),
                                preferred_element_type=jnp.float32)
        diags.append(jnp.sum(power * eye, axis=1, keepdims=True))
    out_ref[0] = jnp.concatenate(diags, axis=1)


def _layernorm(h, s, b):
    m = jnp.mean(h, axis=-1, keepdims=True)
    v = jnp.mean((h - m) * (h - m), axis=-1, keepdims=True)
    return (h - m) / jnp.sqrt(v + 1e-5) * s + b


def _encoder_kernel(x_ref, deg_ref, rw_ref, bias_ref,
                    win_ref, bin_ref, demb_ref, rww_ref, rwb_ref,
                    wq_ref, bq_ref, wk_ref, bk_ref, wv_ref, bv_ref,
                    wo_ref, bo_ref, ln1s_ref, ln1b_ref, ln2s_ref, ln2b_ref,
                    w1_ref, b1_ref, w2_ref, b2_ref, out_ref):
    x = x_ref[0]
    deg = deg_ref[0]
    rw = rw_ref[0]
    h = x @ win_ref[...] + bin_ref[...]
    deg_oh = (deg == lax.broadcasted_iota(jnp.int32, (NPG, MAX_DEG), 1)).astype(jnp.float32)
    h = h + deg_oh @ demb_ref[...]
    h = h + lax.dot_general(rw, rww_ref[...], (((1,), (0,)), ((), ())),
                            preferred_element_type=jnp.float32) + rwb_ref[...]
    for l in range(LAYERS):
        q = h @ wq_ref[l] + bq_ref[l]
        k = h @ wk_ref[l] + bk_ref[l]
        v = h @ wv_ref[l] + bv_ref[l]
        outs = []
        for hh in range(HEADS):
            qh = q[:, hh * HD:(hh + 1) * HD]
            kh = k[:, hh * HD:(hh + 1) * HD]
            vh = v[:, hh * HD:(hh + 1) * HD]
            s = lax.dot_general(qh, kh, (((1,), (1,)), ((), ())),
                                preferred_element_type=jnp.float32) * SCALE
            s = s + bias_ref[0, l * HEADS + hh]
            m = jnp.max(s, axis=1, keepdims=True)
            p = jnp.exp(s - m)
            p = p / jnp.sum(p, axis=1, keepdims=True)
            outs.append(lax.dot_general(p, vh, (((1,), (0,)), ((), ())),
                                        preferred_element_type=jnp.float32))
        attn = jnp.concatenate(outs, axis=1)
        h = _layernorm(h + attn @ wo_ref[l] + bo_ref[l], ln1s_ref[l], ln1b_ref[l])
        ff = jnp.maximum(h @ w1_ref[l] + b1_ref[l], 0.0) @ w2_ref[l] + b2_ref[l]
        h = _layernorm(h + ff, ln2s_ref[l], ln2b_ref[l])
    out_ref[0] = h


def _wid():
    return lax.axis_index("s") * NC + lax.axis_index("c")


def _scal(offv, i):
    return offv[pl.ds(i, 16)][0]


def _zero_ref(ref, nwords):
    z = jnp.zeros((16,), jnp.float32)

    def body(j, _):
        ref[pl.ds(j * 16, 16)] = z
        return 0

    lax.fori_loop(0, nwords // 16, body, 0)


def _reorder_body(rec_hbm, order_hbm, spos_hbm, sebt_hbm, ordv, recv, posv, colv, sem):
    w = _wid()

    def chunk(c, _):
        base = pl.multiple_of(w * EPW + c * CHA, CHA)
        pltpu.sync_copy(order_hbm.at[pl.ds(pl.multiple_of(base // SUB, 8), CHA // SUB)], ordv)
        copies = []
        for k in range(CHA // SUB):
            copies.append(pltpu.async_copy(
                rec_hbm.at[ordv.at[k]], recv.at[pl.ds(k * SUB, SUB)], sem))
        for cp in copies:
            cp.wait()

        def tbody(j, _):
            row = lax.iota(jnp.int32, 16) + j * 16
            posf = plsc.load_gather(recv, [row, jnp.zeros((16,), jnp.int32)])
            posv[pl.ds(j * 16, 16)] = plsc.bitcast(posf, jnp.int32)
            for s in range(NSLOT):
                v = plsc.load_gather(recv, [row, jnp.full((16,), s + 1, jnp.int32)])
                colv[pl.ds(s * CHA + j * 16, 16)] = v
            return 0

        lax.fori_loop(0, CHA // 16, tbody, 0)
        pltpu.sync_copy(posv, spos_hbm.at[pl.ds(base, CHA)])
        for s in range(NSLOT):
            pltpu.sync_copy(colv.at[pl.ds(s * CHA, CHA)],
                            sebt_hbm.at[s, pl.ds(base, CHA)])
        return 0

    lax.fori_loop(0, EPW // CHA, chunk, 0)


def _bias_body(spos_hbm, sebt_hbm, off_hbm, bias_hbm, dest, posv, valv, offv):
    w = _wid()
    pltpu.sync_copy(off_hbm, offv)
    lanes16 = lax.iota(jnp.int32, 16)
    zeros16 = jnp.zeros((16,), jnp.float32)
    _zero_ref(dest, GSIZE)

    def task(t, _):
        tid = w + NW * t
        g = tid // NSLOT
        s = tid % NSLOT
        o0 = _scal(offv, g)
        o1 = _scal(offv, g + 1)
        cnt = o1 - o0
        nch = (cnt + CH - 1) // CH

        def cbody(c, _):
            lo = o0 + c * CH
            hi = jnp.minimum(lo + CH, o1)
            astart = pl.multiple_of((lo // 8) * 8, 8)
            pltpu.sync_copy(spos_hbm.at[pl.ds(astart, CHP)], posv)
            pltpu.sync_copy(sebt_hbm.at[s, pl.ds(astart, CHP)], valv)

            def vbody(j, _):
                gidx = lanes16 + (astart + j * 16)
                mask = (gidx >= lo) & (gidx < hi)
                pv = posv[pl.ds(j * 16, 16)]
                vv = valv[pl.ds(j * 16, 16)]
                plsc.addupdate_scatter(dest, [pv], vv, mask=mask)
                return 0

            lax.fori_loop(0, CHP // 16, vbody, 0)
            return 0

        lax.fori_loop(0, nch, cbody, 0)
        pltpu.sync_copy(dest, bias_hbm.at[g, s])

        def clean(c, _):
            lo = o0 + c * CH
            hi = jnp.minimum(lo + CH, o1)
            astart = pl.multiple_of((lo // 8) * 8, 8)
            pltpu.sync_copy(spos_hbm.at[pl.ds(astart, CHP)], posv)

            def vz(j, _):
                gidx = lanes16 + (astart + j * 16)
                mask = (gidx >= lo) & (gidx < hi)
                pv = posv[pl.ds(j * 16, 16)]
                plsc.store_scatter(dest, [pv], zeros16, mask=mask)
                return 0

            lax.fori_loop(0, CHP // 16, vz, 0)
            return 0

        lax.fori_loop(0, nch, clean, 0)
        return 0

    lax.fori_loop(0, B * NSLOT // NW, task, 0)


def _adj_deg_body(spos_hbm, off_hbm, adj_hbm, deg_hbm, dest, degd, posv, offv):
    w = _wid()
    pltpu.sync_copy(off_hbm, offv)
    lanes16 = lax.iota(jnp.int32, 16)
    ones = jnp.ones((16,), jnp.float32)

    def task(t, _):
        g = w + NW * t
        o0 = _scal(offv, g)
        o1 = _scal(offv, g + 1)
        cnt = o1 - o0
        _zero_ref(dest, GSIZE)
        _zero_ref(degd, NPG)
        nch = (cnt + CH - 1) // CH

        def cbody(c, _):
            lo = o0 + c * CH
            hi = jnp.minimum(lo + CH, o1)
            astart = pl.multiple_of((lo // 8) * 8, 8)
            pltpu.sync_copy(spos_hbm.at[pl.ds(astart, CHP)], posv)

            def vbody(j, _):
                gidx = lanes16 + (astart + j * 16)
                mask = (gidx >= lo) & (gidx < hi)
                pv = posv[pl.ds(j * 16, 16)]
                ptv = ((pv & 255) << 8) | (pv >> 8)
                plsc.addupdate_scatter(dest, [pv], ones, mask=mask)
                plsc.addupdate_scatter(dest, [ptv], ones, mask=mask)
                plsc.addupdate_scatter(degd, [pv & 255], ones, mask=mask)
                return 0

            lax.fori_loop(0, CHP // 16, vbody, 0)
            return 0

        lax.fori_loop(0, nch, cbody, 0)
        pltpu.sync_copy(dest, adj_hbm.at[g])
        pltpu.sync_copy(degd, deg_hbm.at[g])
        return 0

    lax.fori_loop(0, B // NW, task, 0)


_reorder_call = functools.partial(
    pl.kernel,
    out_type=(jax.ShapeDtypeStruct((EPAD,), jnp.int32),
              jax.ShapeDtypeStruct((NSLOT, EPAD), jnp.float32)),
    mesh=_MESH,
    compiler_params=pltpu.CompilerParams(use_tc_tiling_on_sc=False, needs_layout_passes=False),
    scratch_types=[
        pltpu.VMEM((CHA // SUB, SUB), jnp.int32),
        pltpu.VMEM((CHA, 32), jnp.float32),
        pltpu.VMEM((CHA,), jnp.int32),
        pltpu.VMEM((NSLOT * CHA,), jnp.float32),
        pltpu.SemaphoreType.DMA,
    ],
)(_reorder_body)

_bias_call = functools.partial(
    pl.kernel,
    out_type=jax.ShapeDtypeStruct((B, NSLOT, GSIZE), jnp.float32),
    mesh=_MESH,
    compiler_params=pltpu.CompilerParams(use_tc_tiling_on_sc=False, needs_layout_passes=False),
    scratch_types=[
        pltpu.VMEM((GSIZE,), jnp.float32),
        pltpu.VMEM((CHP,), jnp.int32),
        pltpu.VMEM((CHP,), jnp.float32),
        pltpu.VMEM((80,), jnp.int32),
    ],
)(_bias_body)

_adj_deg_call = functools.partial(
    pl.kernel,
    out_type=(jax.ShapeDtypeStruct((B, GSIZE), jnp.float32),
              jax.ShapeDtypeStruct((B, NPG), jnp.float32)),
    mesh=_MESH,
    compiler_params=pltpu.CompilerParams(use_tc_tiling_on_sc=False, needs_layout_passes=False),
    scratch_types=[
        pltpu.VMEM((GSIZE,), jnp.float32),
        pltpu.VMEM((NPG,), jnp.float32),
        pltpu.VMEM((CHP,), jnp.int32),
        pltpu.VMEM((80,), jnp.int32),
    ],
)(_adj_deg_body)


def _whole(shape):
    nd = len(shape)
    return pl.BlockSpec(shape, lambda b, _nd=nd: (0,) * _nd)


def kernel(x, edge_index, edge_attr, edge_gate_type, batch, params):
    p = params
    src = edge_index[0]
    dst = edge_index[1]
    g = src // NPG
    ls = src % NPG
    ld = dst % NPG
    pos = ls * NPG + ld

    gate_cat = jnp.concatenate([p['gate_emb'][0], p['gate_emb'][1]], axis=1)
    ebw_cat = jnp.concatenate([p['ebW'][0], p['ebW'][1]], axis=1)
    ebb_cat = jnp.concatenate([p['ebb'][0], p['ebb'][1]], axis=0).reshape(1, NSLOT)
    rec = pl.pallas_call(
        _edge_rec_kernel,
        grid=(E // EB_BLK,),
        in_specs=[
            pl.BlockSpec((EB_BLK, 1), lambda i: (i, 0)),
            pl.BlockSpec((EB_BLK, 1), lambda i: (i, 0)),
            pl.BlockSpec((EB_BLK, EDGE_FEAT), lambda i: (i, 0)),
            _whole((GATE_TYPES, NSLOT)),
            _whole((EDGE_FEAT, NSLOT)),
            _whole((1, NSLOT)),
        ],
        out_specs=pl.BlockSpec((EB_BLK, 32), lambda i: (i, 0)),
        out_shape=jax.ShapeDtypeStruct((E, 32), jnp.float32),
    )(edge_gate_type.reshape(E, 1), pos.reshape(E, 1), edge_attr,
      gate_cat, ebw_cat, ebb_cat)

    gpos = g * GSIZE + pos
    sorted_gpos, order = lax.sort((gpos, jnp.arange(E, dtype=jnp.int32)), num_keys=1)
    off = jnp.searchsorted(sorted_gpos, jnp.arange(B + 1, dtype=jnp.int32) * GSIZE
                           ).astype(jnp.int32)
    off = jnp.concatenate([off, jnp.zeros((80 - (B + 1),), jnp.int32)])

    spos, sebt = _reorder_call(rec, order.reshape(E // SUB, SUB))
    bias = _bias_call(spos, sebt, off)
    adjs, degc = _adj_deg_call(spos, off)

    deg_idx = jnp.clip(degc.astype(jnp.int32), 0, MAX_DEG - 1).reshape(B, NPG, 1)

    rw = pl.pallas_call(
        _rw_pe_kernel,
        grid=(B,),
        in_specs=[pl.BlockSpec((1, NPG, NPG), lambda b: (b, 0, 0))],
        out_specs=pl.BlockSpec((1, NPG, RW_K), lambda b: (b, 0, 0)),
        out_shape=jax.ShapeDtypeStruct((B, NPG, RW_K), jnp.float32),
    )(adjs.reshape(B, NPG, NPG))

    x3 = x.reshape(B, NPG, NODE_FEAT)
    bias4 = bias.reshape(B, NSLOT, NPG, NPG)
    out = pl.pallas_call(
        _encoder_kernel,
        grid=(B,),
        in_specs=[
            pl.BlockSpec((1, NPG, NODE_FEAT), lambda b: (b, 0, 0)),
            pl.BlockSpec((1, NPG, 1), lambda b: (b, 0, 0)),
            pl.BlockSpec((1, NPG, RW_K), lambda b: (b, 0, 0)),
            pl.BlockSpec((1, NSLOT, NPG, NPG), lambda b: (b, 0, 0, 0)),
            _whole((NODE_FEAT, HID)),
            _whole((1, HID)),
            _whole((MAX_DEG, HID)),
            _whole((RW_K, HID)),
            _whole((1, HID)),
            _whole((LAYERS, HID, HID)),
            _whole((LAYERS, 1, HID)),
            _whole((LAYERS, HID, HID)),
            _whole((LAYERS, 1, HID)),
            _whole((LAYERS, HID, HID)),
            _whole((LAYERS, 1, HID)),
            _whole((LAYERS, HID, HID)),
            _whole((LAYERS, 1, HID)),
            _whole((LAYERS, 1, HID)),
            _whole((LAYERS, 1, HID)),
            _whole((LAYERS, 1, HID)),
            _whole((LAYERS, 1, HID)),
            _whole((LAYERS, HID, FF)),
            _whole((LAYERS, 1, FF)),
            _whole((LAYERS, FF, HID)),
            _whole((LAYERS, 1, HID)),
        ],
        out_specs=pl.BlockSpec((1, NPG, HID), lambda b: (b, 0, 0)),
        out_shape=jax.ShapeDtypeStruct((B, NPG, HID), jnp.float32),
    )(x3, deg_idx, rw, bias4,
      p['Win'], p['bin'].reshape(1, HID), p['deg_emb'], p['rwW'], p['rwb'].reshape(1, HID),
      p['Wq'], p['bq'].reshape(LAYERS, 1, HID), p['Wk'], p['bk'].reshape(LAYERS, 1, HID),
      p['Wv'], p['bv'].reshape(LAYERS, 1, HID), p['Wo'], p['bo'].reshape(LAYERS, 1, HID),
      p['ln1s'].reshape(LAYERS, 1, HID), p['ln1b'].reshape(LAYERS, 1, HID),
      p['ln2s'].reshape(LAYERS, 1, HID), p['ln2b'].reshape(LAYERS, 1, HID),
      p['W1'], p['b1'].reshape(LAYERS, 1, FF), p['W2'], p['b2'].reshape(LAYERS, 1, HID))
    return out.reshape(N, HID)
